# baseline (speedup 1.0000x reference)
import jax
import jax.numpy as jnp
from jax import lax
from jax.experimental import pallas as pl
from jax.experimental.pallas import tpu as pltpu
from jax.experimental.pallas import tpu_sc as plsc

N_USERS = 50000
N_ITEMS = 50000
NR = N_USERS + N_ITEMS
D = 32
DH = D // 2

NC, NS, L = 2, 16, 16
NRP = 100096
ZS = NRP // NS
WLAST = NR - (NS - 1) * ZS

K = 5
CHUNK = K * 128
N_EDGES = 1600000
NQ = 4
NCHUNK = 160
EPT = NCHUNK * CHUNK
E_PAD = NS * EPT


def _sc_body(e0_hbm, pk_hbm, zro_hbm, o1_hbm, o2_hbm, o3_hbm,
             acc_sh, pk_v, msg_v,
             isem0, isem1, isem2, isem3, gsem0, gsem1, ssem0, ssem1):
    c = lax.axis_index("c")
    s = lax.axis_index("s")
    isems = (isem0, isem1, isem2, isem3)
    gsems = (gsem0, gsem1)
    ssems = (ssem0, ssem1)

    def one_layer(in3, out3):
        in_ref = in3.at[c]
        out_ref = out3.at[c]

        z0 = s * ZS
        pltpu.sync_copy(zro_hbm, acc_sh.at[pl.ds(z0, ZS)])
        plsc.subcore_barrier()

        def idx_issue(ch, q):
            pltpu.async_copy(pk_hbm.at[s * NCHUNK + ch], pk_v.at[q],
                             isems[q])

        def idx_wait(ch, q):
            pltpu.make_async_copy(pk_hbm.at[s * NCHUNK + ch], pk_v.at[q],
                                  isems[q]).wait()

        def gather_issue(b, q):
            pltpu.async_copy(in_ref.at[pk_v.at[q].at[0]], msg_v.at[b],
                             gsems[b])

        def gather_wait(b, q):
            pltpu.make_async_copy(in_ref.at[pk_v.at[q].at[0]], msg_v.at[b],
                                  gsems[b]).wait()

        def multiply(b, q):
            @pl.loop(0, CHUNK // L)
            def _(g):
                w16 = plsc.bitcast(pk_v[q, 2, pl.ds(g * L, L)], jnp.float32)
                for r in range(L):
                    i = g * L + r
                    msg_v[b, i, pl.ds(0, L)] = (
                        msg_v[b, i, pl.ds(0, L)] * w16[r])

        def scatter_issue(b, q):
            pltpu.async_copy(msg_v.at[b], acc_sh.at[pk_v.at[q].at[1]],
                             ssems[b], add=True)

        def scatter_wait(b, q):
            pltpu.make_async_copy(msg_v.at[b], acc_sh.at[pk_v.at[q].at[1]],
                                  ssems[b]).wait()

        idx_issue(0, 0)
        idx_issue(1, 1)
        idx_issue(2, 2)
        idx_wait(0, 0)
        gather_issue(0, 0)

        QN = NCHUNK // NQ

        @pl.loop(0, QN)
        def _(it):
            for sub in range(NQ):
                ch = it * NQ + sub
                b, bp = sub % 2, 1 - sub % 2
                q, qn, qp = sub, (sub + 1) % NQ, (sub - 1) % NQ

                gather_wait(b, q)

                if sub == 0:
                    @pl.when(it > 0)
                    def _():
                        scatter_wait(bp, qp)
                else:
                    scatter_wait(bp, qp)

                if sub < NQ - 1:
                    idx_wait(ch + 1, qn)
                    gather_issue(bp, qn)
                else:
                    @pl.when(it < QN - 1)
                    def _():
                        idx_wait(ch + 1, qn)
                        gather_issue(bp, qn)

                multiply(b, q)
                scatter_issue(b, q)

                if sub == 0:
                    idx_issue(ch + 3, (sub + 3) % NQ)
                else:
                    @pl.when(it < QN - 1)
                    def _():
                        idx_issue(ch + 3, (sub + 3) % NQ)

        scatter_wait(1, 3)
        plsc.subcore_barrier()

        w0 = s * ZS

        @pl.when(s < NS - 1)
        def _():
            pltpu.sync_copy(acc_sh.at[pl.ds(w0, ZS)],
                            out_ref.at[pl.ds(w0, ZS)])

        @pl.when(s == NS - 1)
        def _():
            pltpu.sync_copy(acc_sh.at[pl.ds(w0, WLAST)],
                            out_ref.at[pl.ds(w0, WLAST)])

        plsc.subcore_barrier()

    one_layer(e0_hbm, o1_hbm)
    one_layer(o1_hbm, o2_hbm)
    one_layer(o2_hbm, o3_hbm)


def _make_sc_kernel():
    mesh = plsc.VectorSubcoreMesh(core_axis_name="c", subcore_axis_name="s",
                                  num_cores=NC, num_subcores=NS)
    half = jax.ShapeDtypeStruct((NC, NR, DH), jnp.float32)
    return pl.kernel(
        _sc_body,
        out_type=(half, half, half),
        mesh=mesh,
        scratch_types=[
            pltpu.VMEM_SHARED((NRP, DH), jnp.float32),
            pltpu.VMEM((NQ, 3, CHUNK), jnp.int32),
            pltpu.VMEM((2, CHUNK, DH), jnp.float32),
        ] + [pltpu.SemaphoreType.DMA] * 8,
        compiler_params=pltpu.CompilerParams(use_tc_tiling_on_sc=False,
                                             needs_layout_passes=False),
    )


def _mean_body(a_ref, b_ref, c_ref, d_ref, o_ref):
    o_ref[0] = (a_ref[0] + b_ref[0] + c_ref[0] + d_ref[0]) * 0.25


def kernel(user_emb, item_emb, edge_index, edge_weight):
    e0 = jnp.concatenate([user_emb, item_emb], axis=0)
    e0_st = jnp.stack([e0[:, :DH], e0[:, DH:]], axis=0)
    pad = E_PAD - N_EDGES
    src = jnp.pad(edge_index[0], (0, pad)).reshape(NS * NCHUNK, CHUNK)
    dst = jnp.pad(edge_index[1], (0, pad)).reshape(NS * NCHUNK, CHUNK)
    wb = lax.bitcast_convert_type(jnp.pad(edge_weight, (0, pad)),
                                  jnp.int32).reshape(NS * NCHUNK, CHUNK)
    pk = jnp.stack([src, dst, wb], axis=1)
    zro = jnp.zeros((ZS, DH), jnp.float32)

    o1, o2, o3 = _make_sc_kernel()(e0_st, pk, zro)

    blk = 800
    zs = pl.pallas_call(
        _mean_body,
        out_shape=jax.ShapeDtypeStruct((NC, NR, DH), jnp.float32),
        grid=(NC, NR // blk),
        in_specs=[pl.BlockSpec((1, blk, DH), lambda i, j: (i, j, 0))] * 4,
        out_specs=pl.BlockSpec((1, blk, DH), lambda i, j: (i, j, 0)),
    )(e0_st, o1, o2, o3)

    z = jnp.concatenate([zs[0], zs[1]], axis=1)
    return z[:N_USERS], z[N_USERS:]


# speedup vs baseline: 1.0098x; 1.0098x over previous
import jax
import jax.numpy as jnp
from jax import lax
from jax.experimental import pallas as pl
from jax.experimental.pallas import tpu as pltpu
from jax.experimental.pallas import tpu_sc as plsc

N_USERS = 50000
N_ITEMS = 50000
NR = N_USERS + N_ITEMS
D = 32
DH = D // 2

NC, NS, L = 2, 16, 16
NRP = 100096
ZS = NRP // NS
WLAST = NR - (NS - 1) * ZS

K = 3
CHUNK = K * 128
N_EDGES = 1600000
NB = 3
NQ = 6
NCHUNK = 264
EPT = NCHUNK * CHUNK
E_PAD = NS * EPT


def _sc_body(e0_hbm, pk_hbm, zro_hbm, o1_hbm, o2_hbm, o3_hbm,
             acc_sh, pk_v, msg_v,
             isem0, isem1, isem2, isem3, isem4, isem5,
             gsem0, gsem1, gsem2, ssem0, ssem1, ssem2):
    c = lax.axis_index("c")
    s = lax.axis_index("s")
    isems = (isem0, isem1, isem2, isem3, isem4, isem5)
    gsems = (gsem0, gsem1, gsem2)
    ssems = (ssem0, ssem1, ssem2)

    def one_layer(in3, out3):
        in_ref = in3.at[c]
        out_ref = out3.at[c]

        z0 = s * ZS
        pltpu.sync_copy(zro_hbm, acc_sh.at[pl.ds(z0, ZS)])
        plsc.subcore_barrier()

        def idx_issue(ch, q):
            pltpu.async_copy(pk_hbm.at[s * NCHUNK + ch], pk_v.at[q],
                             isems[q])

        def idx_wait(ch, q):
            pltpu.make_async_copy(pk_hbm.at[s * NCHUNK + ch], pk_v.at[q],
                                  isems[q]).wait()

        def gather_issue(b, q):
            for j in range(K):
                pltpu.async_copy(in_ref.at[pk_v.at[q].at[0].at[j]],
                                 msg_v.at[b].at[pl.ds(j * 128, 128)],
                                 gsems[b])

        def gather_wait(b, q):
            for j in range(K):
                pltpu.make_async_copy(in_ref.at[pk_v.at[q].at[0].at[j]],
                                      msg_v.at[b].at[pl.ds(j * 128, 128)],
                                      gsems[b]).wait()

        def multiply(b, q):
            for j in range(K):
                @pl.loop(0, 128 // L)
                def _(g, j=j):
                    w16 = plsc.bitcast(pk_v[q, 2, j, pl.ds(g * L, L)],
                                       jnp.float32)
                    for r in range(L):
                        i = j * 128 + g * L + r
                        msg_v[b, i, pl.ds(0, L)] = (
                            msg_v[b, i, pl.ds(0, L)] * w16[r])

        def scatter_issue(b, q):
            for j in range(K):
                pltpu.async_copy(msg_v.at[b].at[pl.ds(j * 128, 128)],
                                 acc_sh.at[pk_v.at[q].at[1].at[j]],
                                 ssems[b], add=True)

        def scatter_wait(b, q):
            for j in range(K):
                pltpu.make_async_copy(msg_v.at[b].at[pl.ds(j * 128, 128)],
                                      acc_sh.at[pk_v.at[q].at[1].at[j]],
                                      ssems[b]).wait()

        idx_issue(0, 0)
        idx_issue(1, 1)
        idx_issue(2, 2)
        idx_wait(0, 0)
        gather_issue(0, 0)

        QN = NCHUNK // NQ

        @pl.loop(0, QN)
        def _(it):
            for sub in range(NQ):
                ch = it * NQ + sub
                b, bn = sub % NB, (sub + 1) % NB
                bd = (sub - 2) % NB
                q, qn = sub, (sub + 1) % NQ
                qd = (sub - 2) % NQ

                gather_wait(b, q)

                if sub < 2:
                    @pl.when(it > 0)
                    def _():
                        scatter_wait(bd, qd)
                else:
                    scatter_wait(bd, qd)

                if sub < NQ - 1:
                    idx_wait(ch + 1, qn)
                    gather_issue(bn, qn)
                else:
                    @pl.when(it < QN - 1)
                    def _():
                        idx_wait(ch + 1, qn)
                        gather_issue(bn, qn)

                multiply(b, q)
                scatter_issue(b, q)

                if sub < NQ - 3:
                    idx_issue(ch + 3, (sub + 3) % NQ)
                else:
                    @pl.when(it < QN - 1)
                    def _():
                        idx_issue(ch + 3, (sub + 3) % NQ)

        scatter_wait((NCHUNK - 2) % NB, (NCHUNK - 2) % NQ)
        scatter_wait((NCHUNK - 1) % NB, (NCHUNK - 1) % NQ)
        plsc.subcore_barrier()

        w0 = s * ZS

        @pl.when(s < NS - 1)
        def _():
            pltpu.sync_copy(acc_sh.at[pl.ds(w0, ZS)],
                            out_ref.at[pl.ds(w0, ZS)])

        @pl.when(s == NS - 1)
        def _():
            pltpu.sync_copy(acc_sh.at[pl.ds(w0, WLAST)],
                            out_ref.at[pl.ds(w0, WLAST)])

        plsc.subcore_barrier()

    one_layer(e0_hbm, o1_hbm)
    one_layer(o1_hbm, o2_hbm)
    one_layer(o2_hbm, o3_hbm)


def _make_sc_kernel():
    mesh = plsc.VectorSubcoreMesh(core_axis_name="c", subcore_axis_name="s",
                                  num_cores=NC, num_subcores=NS)
    half = jax.ShapeDtypeStruct((NC, NR, DH), jnp.float32)
    return pl.kernel(
        _sc_body,
        out_type=(half, half, half),
        mesh=mesh,
        scratch_types=[
            pltpu.VMEM_SHARED((NRP, DH), jnp.float32),
            pltpu.VMEM((NQ, 3, K, 128), jnp.int32),
            pltpu.VMEM((NB, CHUNK, DH), jnp.float32),
        ] + [pltpu.SemaphoreType.DMA] * 12,
        compiler_params=pltpu.CompilerParams(use_tc_tiling_on_sc=False,
                                             needs_layout_passes=False),
    )


def _mean_body(a_ref, b_ref, c_ref, d_ref, o_ref):
    o_ref[0] = (a_ref[0] + b_ref[0] + c_ref[0] + d_ref[0]) * 0.25


def kernel(user_emb, item_emb, edge_index, edge_weight):
    e0 = jnp.concatenate([user_emb, item_emb], axis=0)
    e0_st = jnp.stack([e0[:, :DH], e0[:, DH:]], axis=0)
    pad = E_PAD - N_EDGES
    src = jnp.pad(edge_index[0], (0, pad)).reshape(NS * NCHUNK, K, 128)
    dst = jnp.pad(edge_index[1], (0, pad)).reshape(NS * NCHUNK, K, 128)
    wb = lax.bitcast_convert_type(jnp.pad(edge_weight, (0, pad)),
                                  jnp.int32).reshape(NS * NCHUNK, K, 128)
    pk = jnp.stack([src, dst, wb], axis=1)
    zro = jnp.zeros((ZS, DH), jnp.float32)

    o1, o2, o3 = _make_sc_kernel()(e0_st, pk, zro)

    blk = 800
    zs = pl.pallas_call(
        _mean_body,
        out_shape=jax.ShapeDtypeStruct((NC, NR, DH), jnp.float32),
        grid=(NC, NR // blk),
        in_specs=[pl.BlockSpec((1, blk, DH), lambda i, j: (i, j, 0))] * 4,
        out_specs=pl.BlockSpec((1, blk, DH), lambda i, j: (i, j, 0)),
    )(e0_st, o1, o2, o3)

    z = jnp.concatenate([zs[0], zs[1]], axis=1)
    return z[:N_USERS], z[N_USERS:]


# speedup vs baseline: 1.2816x; 1.2692x over previous
import jax
import jax.numpy as jnp
from jax import lax
from jax.experimental import pallas as pl
from jax.experimental.pallas import tpu as pltpu
from jax.experimental.pallas import tpu_sc as plsc

N_USERS = 50000
N_ITEMS = 50000
NR = N_USERS + N_ITEMS
D = 32
DH = D // 2

NC, NS, L = 2, 16, 16
NRP = 100096
ZS = NRP // NS
WLAST = NR - (NS - 1) * ZS

K = 4
CHUNK = K * 128
N_EDGES = 1600000
NCHUNK = -(-N_EDGES // (NS * CHUNK))
EPT = NCHUNK * CHUNK
E_PAD = NS * EPT
NQ = 4


def _sc_body(e0_hbm, src_hbm, dst_hbm, w_hbm, zro_hbm, o1_hbm, o2_hbm, o3_hbm,
             acc_sh, src_v, dst_v, w_v, msg_v,
             isem0, isem1, isem2, isem3, gsem0, gsem1, ssem0, ssem1):
    c = lax.axis_index("c")
    s = lax.axis_index("s")
    isems = (isem0, isem1, isem2, isem3)
    gsems = (gsem0, gsem1)
    ssems = (ssem0, ssem1)

    def one_layer(in3, out3):
        in_ref = in3.at[c]
        out_ref = out3.at[c]

        z0 = s * ZS
        pltpu.sync_copy(zro_hbm, acc_sh.at[pl.ds(z0, ZS)])
        plsc.subcore_barrier()

        def idx_issue(ch, q):
            row0 = s * (EPT // 128) + ch * K
            flat0 = s * EPT + ch * CHUNK
            pltpu.async_copy(src_hbm.at[pl.ds(row0, K)], src_v.at[q],
                             isems[q])
            pltpu.async_copy(dst_hbm.at[pl.ds(row0, K)], dst_v.at[q],
                             isems[q])
            pltpu.async_copy(w_hbm.at[pl.ds(flat0, CHUNK)], w_v.at[q],
                             isems[q])

        def idx_wait(ch, q):
            row0 = s * (EPT // 128) + ch * K
            flat0 = s * EPT + ch * CHUNK
            pltpu.make_async_copy(src_hbm.at[pl.ds(row0, K)], src_v.at[q],
                                  isems[q]).wait()
            pltpu.make_async_copy(dst_hbm.at[pl.ds(row0, K)], dst_v.at[q],
                                  isems[q]).wait()
            pltpu.make_async_copy(w_hbm.at[pl.ds(flat0, CHUNK)], w_v.at[q],
                                  isems[q]).wait()

        def gather_issue(b, q):
            for j in range(K):
                pltpu.async_copy(in_ref.at[src_v.at[q].at[j]],
                                 msg_v.at[b].at[pl.ds(j * 128, 128)],
                                 gsems[b])

        def gather_wait(b, q):
            for j in range(K):
                pltpu.make_async_copy(in_ref.at[src_v.at[q].at[j]],
                                      msg_v.at[b].at[pl.ds(j * 128, 128)],
                                      gsems[b]).wait()

        def multiply(b, q):
            @pl.loop(0, CHUNK // L)
            def _(g):
                w16 = w_v[q, pl.ds(g * L, L)]
                for r in range(L):
                    i = g * L + r
                    msg_v[b, i, pl.ds(0, L)] = msg_v[b, i, pl.ds(0, L)] * w16[r]

        def scatter_issue(b, q):
            for j in range(K):
                pltpu.async_copy(msg_v.at[b].at[pl.ds(j * 128, 128)],
                                 acc_sh.at[dst_v.at[q].at[j]],
                                 ssems[b], add=True)

        def scatter_wait(b, q):
            for j in range(K):
                pltpu.make_async_copy(msg_v.at[b].at[pl.ds(j * 128, 128)],
                                      acc_sh.at[dst_v.at[q].at[j]],
                                      ssems[b]).wait()

        idx_issue(0, 0)
        idx_issue(1, 1)
        idx_issue(2, 2)
        idx_wait(0, 0)
        gather_issue(0, 0)

        QN = NCHUNK // NQ

        @pl.loop(0, QN)
        def _(it):
            for sub in range(NQ):
                ch = it * NQ + sub
                b, bp = sub % 2, 1 - sub % 2
                q, qn, qp = sub, (sub + 1) % NQ, (sub - 1) % NQ

                gather_wait(b, q)

                if sub == 0:
                    @pl.when(it > 0)
                    def _():
                        scatter_wait(bp, qp)
                else:
                    scatter_wait(bp, qp)

                if sub < NQ - 1:
                    idx_wait(ch + 1, qn)
                    gather_issue(bp, qn)
                else:
                    @pl.when(it < QN - 1)
                    def _():
                        idx_wait(ch + 1, qn)
                        gather_issue(bp, qn)

                multiply(b, q)
                scatter_issue(b, q)

                if sub == 0:
                    idx_issue(ch + 3, (sub + 3) % NQ)
                else:
                    @pl.when(it < QN - 1)
                    def _():
                        idx_issue(ch + 3, (sub + 3) % NQ)

        scatter_wait(1, 3)
        plsc.subcore_barrier()

        w0 = s * ZS

        @pl.when(s < NS - 1)
        def _():
            pltpu.sync_copy(acc_sh.at[pl.ds(w0, ZS)],
                            out_ref.at[pl.ds(w0, ZS)])

        @pl.when(s == NS - 1)
        def _():
            pltpu.sync_copy(acc_sh.at[pl.ds(w0, WLAST)],
                            out_ref.at[pl.ds(w0, WLAST)])

        plsc.subcore_barrier()

    one_layer(e0_hbm, o1_hbm)
    one_layer(o1_hbm, o2_hbm)
    one_layer(o2_hbm, o3_hbm)


def _make_sc_kernel():
    mesh = plsc.VectorSubcoreMesh(core_axis_name="c", subcore_axis_name="s",
                                  num_cores=NC, num_subcores=NS)
    half = jax.ShapeDtypeStruct((NC, NR, DH), jnp.float32)
    return pl.kernel(
        _sc_body,
        out_type=(half, half, half),
        mesh=mesh,
        scratch_types=[
            pltpu.VMEM_SHARED((NRP, DH), jnp.float32),
            pltpu.VMEM((NQ, K, 128), jnp.int32),
            pltpu.VMEM((NQ, K, 128), jnp.int32),
            pltpu.VMEM((NQ, CHUNK), jnp.float32),
            pltpu.VMEM((2, CHUNK, DH), jnp.float32),
        ] + [pltpu.SemaphoreType.DMA] * 8,
        compiler_params=pltpu.CompilerParams(use_tc_tiling_on_sc=False),
    )


def _mean_body(a_ref, b_ref, c_ref, d_ref, o_ref):
    o_ref[0] = (a_ref[0] + b_ref[0] + c_ref[0] + d_ref[0]) * 0.25


def kernel(user_emb, item_emb, edge_index, edge_weight):
    e0 = jnp.concatenate([user_emb, item_emb], axis=0)
    e0_st = jnp.stack([e0[:, :DH], e0[:, DH:]], axis=0)
    pad = E_PAD - N_EDGES
    src = jnp.pad(edge_index[0], (0, pad)).reshape(E_PAD // 128, 128)
    dst = jnp.pad(edge_index[1], (0, pad)).reshape(E_PAD // 128, 128)
    w = jnp.pad(edge_weight, (0, pad))
    zro = jnp.zeros((ZS, DH), jnp.float32)

    o1, o2, o3 = _make_sc_kernel()(e0_st, src, dst, w, zro)

    blk = 800
    zs = pl.pallas_call(
        _mean_body,
        out_shape=jax.ShapeDtypeStruct((NC, NR, DH), jnp.float32),
        grid=(NC, NR // blk),
        in_specs=[pl.BlockSpec((1, blk, DH), lambda i, j: (i, j, 0))] * 4,
        out_specs=pl.BlockSpec((1, blk, DH), lambda i, j: (i, j, 0)),
    )(e0_st, o1, o2, o3)

    z = jnp.concatenate([zs[0], zs[1]], axis=1)
    return z[:N_USERS], z[N_USERS:]
